# baseline (speedup 1.0000x reference)
import jax
import jax.numpy as jnp
from jax import lax
from jax.experimental import pallas as pl
from jax.experimental.pallas import tpu as pltpu

_STRIDE = 8.0
_BA = 16384
_G = 64
_R = _BA // 128


def _cumsum_1d(m):
    t = m.reshape(_R, 128)
    r0 = lax.broadcasted_iota(jnp.int32, (128, 128), 0)
    c0 = lax.broadcasted_iota(jnp.int32, (128, 128), 1)
    upper = (r0 <= c0).astype(jnp.float32)
    within = jnp.dot(t, upper, preferred_element_type=jnp.float32)
    rowtot = jnp.sum(t, axis=1)
    rr = lax.broadcasted_iota(jnp.int32, (_R, _R), 0)
    cc = lax.broadcasted_iota(jnp.int32, (_R, _R), 1)
    prefix = jnp.sum(jnp.where(rr < cc, rowtot[:, None], 0.0), axis=0)
    return (within + prefix[:, None]).reshape(m.shape)


def _gt_corners(tgt):
    x0, y0 = tgt[:, 0], tgt[:, 1]
    x1, y1 = tgt[:, 2], tgt[:, 3]
    x2, y2 = tgt[:, 4], tgt[:, 5]
    x3, y3 = tgt[:, 6], tgt[:, 7]
    gx1 = jnp.minimum(jnp.minimum(x0, x1), jnp.minimum(x2, x3))
    gx2 = jnp.maximum(jnp.maximum(x0, x1), jnp.maximum(x2, x3))
    gy1 = jnp.minimum(jnp.minimum(y0, y1), jnp.minimum(y2, y3))
    gy2 = jnp.maximum(jnp.maximum(y0, y1), jnp.maximum(y2, y3))
    return gx1, gy1, gx2, gy2


def _body(off_ref, obj_ref, anc_ref, tgt_ref, ocls_ref, oreg_ref,
          miou_s, selb_s, negm_s, bce_s, bestv_s, besti_s, acc):
    p = pl.program_id(1)
    j = pl.program_id(2)
    offf = (j * _BA).astype(jnp.float32)

    @pl.when(jnp.logical_and(p == 0, j == 0))
    def _init():
        bestv_s[0, :] = jnp.full((_G,), -1.0, jnp.float32)
        besti_s[0, :] = jnp.zeros((_G,), jnp.float32)
        acc[0] = 0.0
        acc[1] = 0.0
        acc[2] = 0.0
        acc[3] = 0.0
        acc[4] = 0.0

    @pl.when(p == 0)
    def _pass0():
        anc = anc_ref[0]
        cx, cy, w, h = anc[0], anc[1], anc[2], anc[3]
        ax1 = cx - w / 2
        ay1 = cy - h / 2
        ax2 = cx + w / 2
        ay2 = cy + h / 2
        gx1, gy1, gx2, gy2 = _gt_corners(tgt_ref[0])
        ltx = jnp.maximum(ax1[None, :], gx1[:, None])
        lty = jnp.maximum(ay1[None, :], gy1[:, None])
        rbx = jnp.minimum(ax2[None, :], gx2[:, None])
        rby = jnp.minimum(ay2[None, :], gy2[:, None])
        iw = jnp.clip(rbx - ltx, 0.0)
        ih = jnp.clip(rby - lty, 0.0)
        inter = iw * ih
        area_a = (ax2 - ax1) * (ay2 - ay1)
        area_g = (gx2 - gx1) * (gy2 - gy1)
        iou = inter / (area_a[None, :] + area_g[:, None] - inter + 1e-9)

        miou = jnp.max(iou, axis=0)
        iog = lax.broadcasted_iota(jnp.int32, (_G, _BA), 0).astype(jnp.float32)
        arg = jnp.min(jnp.where(iou == miou[None, :], iog, 1e9), axis=0)
        miou_s[j] = miou[None, :]
        onehot = (iog == arg[None, :]).astype(jnp.float32)
        tbl = jnp.stack([gx1, gy1, gx2, gy2], axis=0)
        selb_s[j] = jnp.dot(tbl, onehot, precision=lax.Precision.HIGHEST,
                            preferred_element_type=jnp.float32)

        rowmax = jnp.max(iou, axis=1)
        ioa = lax.broadcasted_iota(jnp.int32, (_G, _BA), 1).astype(jnp.float32) + offf
        rowidx = jnp.min(jnp.where(iou == rowmax[:, None], ioa, 1e9), axis=1)
        better = rowmax > bestv_s[0, :]
        bestv_s[0, :] = jnp.where(better, rowmax, bestv_s[0, :])
        besti_s[0, :] = jnp.where(better, rowidx, besti_s[0, :])

    @pl.when(p == 1)
    def _pass1():
        miou = miou_s[j][0]
        ioa = lax.broadcasted_iota(jnp.int32, (_G, _BA), 1).astype(jnp.float32) + offf
        is_best = jnp.any(besti_s[0, :][:, None] == ioa, axis=0)
        pos_mask = jnp.logical_or(miou >= 0.7, is_best)
        posf = pos_mask.astype(jnp.float32)
        base = acc[0]
        rank = _cumsum_1d(posf) + base
        pos_sel = jnp.logical_and(pos_mask, rank <= 128.0)
        self_f = pos_sel.astype(jnp.float32)
        acc[0] = base + jnp.sum(posf)
        acc[1] = acc[1] + jnp.sum(self_f)
        negf = jnp.where(jnp.logical_and(miou < 0.3,
                                         jnp.logical_not(pos_mask)), 1.0, 0.0)
        negm_s[j] = negf[None, :]

        logit = obj_ref[0, 0, 0]
        bce0 = jnp.maximum(logit, 0.0) + jnp.log1p(jnp.exp(-jnp.abs(logit)))
        bce_s[j] = bce0[None, :]
        acc[3] = acc[3] + jnp.sum(jnp.where(pos_sel, bce0 - logit, 0.0))

        selt = selb_s[j]
        bx1 = selt[0] / _STRIDE
        by1 = selt[1] / _STRIDE
        bx2 = selt[2] / _STRIDE
        by2 = selt[3] / _STRIDE
        gx = (bx1 + bx2) / 2
        gy = (by1 + by2) / 2
        gw = bx2 - bx1
        gh = by2 - by1
        anc = anc_ref[0]
        ax, ay, aw, ah = anc[0], anc[1], anc[2], anc[3]
        rt0 = (gx - ax) / aw
        rt1 = (gy - ay) / ah
        rt2 = jnp.log(gw / aw)
        rt3 = jnp.log(gh / ah)
        rt4 = (bx1 - gx) / gw
        rt5 = (by1 - gy) / gh
        off = off_ref[0]
        total = jnp.zeros((_BA,), jnp.float32)
        for k, rt in enumerate((rt0, rt1, rt2, rt3, rt4, rt5)):
            d = off[k] - rt
            ad = jnp.abs(d)
            total = total + jnp.where(ad < 1.0, 0.5 * d * d, ad - 0.5)
        acc[4] = acc[4] + jnp.sum(jnp.where(pos_sel, total, 0.0))

    @pl.when(p == 2)
    def _pass2():
        negf = negm_s[j][0]
        base = acc[2]
        rank = _cumsum_1d(negf) + base
        limit = 256.0 - acc[1]
        neg_sel = jnp.logical_and(negf > 0.0, rank <= limit)
        acc[2] = base + jnp.sum(negf)
        acc[3] = acc[3] + jnp.sum(jnp.where(neg_sel, bce_s[j][0], 0.0))
        ocls_ref[0] = jnp.full((8, 128), acc[3], jnp.float32)
        oreg_ref[0] = jnp.full((8, 128), acc[4], jnp.float32)


def kernel(anchor_offsets, objectness_scores, anchors, targets):
    B, A, _ = anchor_offsets.shape
    nb = A // _BA
    offs_t = jnp.transpose(anchor_offsets, (0, 2, 1))
    anc_t = jnp.transpose(anchors, (0, 2, 1))
    obj_r = objectness_scores.reshape(B, nb, 1, _BA)
    tgt_r = targets.reshape(B, _G, 8)

    out = pl.pallas_call(
        _body,
        grid=(B, 3, nb),
        in_specs=[
            pl.BlockSpec((1, 6, _BA), lambda b, p, j: (b, 0, j)),
            pl.BlockSpec((1, 1, 1, _BA), lambda b, p, j: (b, j, 0, 0)),
            pl.BlockSpec((1, 4, _BA), lambda b, p, j: (b, 0, j)),
            pl.BlockSpec((1, _G, 8), lambda b, p, j: (b, 0, 0)),
        ],
        out_specs=[
            pl.BlockSpec((1, 8, 128), lambda b, p, j: (b, 0, 0)),
            pl.BlockSpec((1, 8, 128), lambda b, p, j: (b, 0, 0)),
        ],
        out_shape=[
            jax.ShapeDtypeStruct((B, 8, 128), jnp.float32),
            jax.ShapeDtypeStruct((B, 8, 128), jnp.float32),
        ],
        scratch_shapes=[
            pltpu.VMEM((nb, 1, _BA), jnp.float32),
            pltpu.VMEM((nb, 4, _BA), jnp.float32),
            pltpu.VMEM((nb, 1, _BA), jnp.float32),
            pltpu.VMEM((nb, 1, _BA), jnp.float32),
            pltpu.VMEM((1, _G), jnp.float32),
            pltpu.VMEM((1, _G), jnp.float32),
            pltpu.SMEM((8,), jnp.float32),
        ],
    )(offs_t, obj_r, anc_t, tgt_r)

    cls_losses = out[0][:, 0, 0]
    regr_losses = out[1][:, 0, 0]
    c = jnp.mean(cls_losses)
    r = jnp.mean(regr_losses)
    return jnp.stack([c + r, c, r])


# speedup vs baseline: 1.1008x; 1.1008x over previous
import jax
import jax.numpy as jnp
from jax import lax
from jax.experimental import pallas as pl
from jax.experimental.pallas import tpu as pltpu

_STRIDE = 8.0
_BA = 16384
_G = 64
_R = _BA // 128


def _cumsum_1d(m):
    t = m.reshape(_R, 128)
    r0 = lax.broadcasted_iota(jnp.int32, (128, 128), 0)
    c0 = lax.broadcasted_iota(jnp.int32, (128, 128), 1)
    upper = (r0 <= c0).astype(jnp.float32)
    within = jnp.dot(t, upper, preferred_element_type=jnp.float32)
    rowtot = jnp.sum(t, axis=1)
    rr = lax.broadcasted_iota(jnp.int32, (_R, _R), 0)
    cc = lax.broadcasted_iota(jnp.int32, (_R, _R), 1)
    prefix = jnp.sum(jnp.where(rr < cc, rowtot[:, None], 0.0), axis=0)
    return (within + prefix[:, None]).reshape(m.shape)


def _gt_corners(tgt):
    x0, y0 = tgt[:, 0], tgt[:, 1]
    x1, y1 = tgt[:, 2], tgt[:, 3]
    x2, y2 = tgt[:, 4], tgt[:, 5]
    x3, y3 = tgt[:, 6], tgt[:, 7]
    gx1 = jnp.minimum(jnp.minimum(x0, x1), jnp.minimum(x2, x3))
    gx2 = jnp.maximum(jnp.maximum(x0, x1), jnp.maximum(x2, x3))
    gy1 = jnp.minimum(jnp.minimum(y0, y1), jnp.minimum(y2, y3))
    gy2 = jnp.maximum(jnp.maximum(y0, y1), jnp.maximum(y2, y3))
    return gx1, gy1, gx2, gy2


def _body(off_ref, obj_ref, anc_ref, tgt_ref, ocls_ref, oreg_ref,
          miou_s, arg_s, negm_s, bce_s, bestv_s, besti_s, acc):
    p = pl.program_id(1)
    j = pl.program_id(2)
    offf = (j * _BA).astype(jnp.float32)

    @pl.when(jnp.logical_and(p == 0, j == 0))
    def _init():
        bestv_s[0, :] = jnp.full((_G,), -1.0, jnp.float32)
        besti_s[0, :] = jnp.zeros((_G,), jnp.float32)
        acc[0] = 0.0
        acc[1] = 0.0
        acc[2] = 0.0
        acc[3] = 0.0
        acc[4] = 0.0

    @pl.when(p == 0)
    def _pass0():
        anc = anc_ref[0]
        cx, cy, w, h = anc[0], anc[1], anc[2], anc[3]
        ax1 = cx - w / 2
        ay1 = cy - h / 2
        ax2 = cx + w / 2
        ay2 = cy + h / 2
        gx1, gy1, gx2, gy2 = _gt_corners(tgt_ref[0])
        ltx = jnp.maximum(ax1[None, :], gx1[:, None])
        lty = jnp.maximum(ay1[None, :], gy1[:, None])
        rbx = jnp.minimum(ax2[None, :], gx2[:, None])
        rby = jnp.minimum(ay2[None, :], gy2[:, None])
        iw = jnp.clip(rbx - ltx, 0.0)
        ih = jnp.clip(rby - lty, 0.0)
        inter = iw * ih
        area_a = (ax2 - ax1) * (ay2 - ay1)
        area_g = (gx2 - gx1) * (gy2 - gy1)
        iou = inter / (area_a[None, :] + area_g[:, None] - inter + 1e-9)

        miou = jnp.max(iou, axis=0)
        iog = lax.broadcasted_iota(jnp.int32, (_G, _BA), 0).astype(jnp.float32)
        arg = jnp.min(jnp.where(iou == miou[None, :], iog, 1e9), axis=0)
        miou_s[j] = miou[None, :]
        arg_s[j] = arg[None, :]

        rowmax = jnp.max(iou, axis=1)
        ioa = lax.broadcasted_iota(jnp.int32, (_G, _BA), 1).astype(jnp.float32) + offf
        rowidx = jnp.min(jnp.where(iou == rowmax[:, None], ioa, 1e9), axis=1)
        better = rowmax > bestv_s[0, :]
        bestv_s[0, :] = jnp.where(better, rowmax, bestv_s[0, :])
        besti_s[0, :] = jnp.where(better, rowidx, besti_s[0, :])

    @pl.when(p == 1)
    def _pass1():
        miou = miou_s[j][0]
        arg = arg_s[j][0]
        ioa = lax.broadcasted_iota(jnp.int32, (_G, _BA), 1).astype(jnp.float32) + offf
        is_best = jnp.any(besti_s[0, :][:, None] == ioa, axis=0)
        pos_mask = jnp.logical_or(miou >= 0.7, is_best)
        posf = pos_mask.astype(jnp.float32)
        base = acc[0]
        rank = _cumsum_1d(posf) + base
        pos_sel = jnp.logical_and(pos_mask, rank <= 128.0)
        self_f = pos_sel.astype(jnp.float32)
        acc[0] = base + jnp.sum(posf)
        acc[1] = acc[1] + jnp.sum(self_f)
        negf = jnp.where(jnp.logical_and(miou < 0.3,
                                         jnp.logical_not(pos_mask)), 1.0, 0.0)
        negm_s[j] = negf[None, :]

        logit = obj_ref[0, 0, 0]
        bce0 = jnp.maximum(logit, 0.0) + jnp.log1p(jnp.exp(-jnp.abs(logit)))
        bce_s[j] = bce0[None, :]
        acc[3] = acc[3] + jnp.sum(jnp.where(pos_sel, bce0 - logit, 0.0))

        gx1, gy1, gx2, gy2 = _gt_corners(tgt_ref[0])
        iog = lax.broadcasted_iota(jnp.int32, (_G, _BA), 0).astype(jnp.float32)
        onehot = (iog == arg[None, :]).astype(jnp.float32)
        tbl = jnp.stack([gx1, gy1, gx2, gy2], axis=0)
        selt = jnp.dot(tbl, onehot, precision=lax.Precision.HIGHEST,
                       preferred_element_type=jnp.float32)
        bx1 = selt[0] / _STRIDE
        by1 = selt[1] / _STRIDE
        bx2 = selt[2] / _STRIDE
        by2 = selt[3] / _STRIDE
        gx = (bx1 + bx2) / 2
        gy = (by1 + by2) / 2
        gw = bx2 - bx1
        gh = by2 - by1
        anc = anc_ref[0]
        ax, ay, aw, ah = anc[0], anc[1], anc[2], anc[3]
        rt0 = (gx - ax) / aw
        rt1 = (gy - ay) / ah
        rt2 = jnp.log(gw / aw)
        rt3 = jnp.log(gh / ah)
        rt4 = (bx1 - gx) / gw
        rt5 = (by1 - gy) / gh
        off = off_ref[0]
        total = jnp.zeros((_BA,), jnp.float32)
        for k, rt in enumerate((rt0, rt1, rt2, rt3, rt4, rt5)):
            d = off[k] - rt
            ad = jnp.abs(d)
            total = total + jnp.where(ad < 1.0, 0.5 * d * d, ad - 0.5)
        acc[4] = acc[4] + jnp.sum(jnp.where(pos_sel, total, 0.0))

    @pl.when(p == 2)
    def _pass2():
        negf = negm_s[j][0]
        base = acc[2]
        rank = _cumsum_1d(negf) + base
        limit = 256.0 - acc[1]
        neg_sel = jnp.logical_and(negf > 0.0, rank <= limit)
        acc[2] = base + jnp.sum(negf)
        acc[3] = acc[3] + jnp.sum(jnp.where(neg_sel, bce_s[j][0], 0.0))
        ocls_ref[0] = jnp.full((8, 128), acc[3], jnp.float32)
        oreg_ref[0] = jnp.full((8, 128), acc[4], jnp.float32)


def kernel(anchor_offsets, objectness_scores, anchors, targets):
    B, A, _ = anchor_offsets.shape
    nb = A // _BA
    offs_t = jnp.transpose(anchor_offsets, (0, 2, 1))
    anc_t = jnp.transpose(anchors, (0, 2, 1))
    obj_r = objectness_scores.reshape(B, nb, 1, _BA)
    tgt_r = targets.reshape(B, _G, 8)

    out = pl.pallas_call(
        _body,
        grid=(B, 3, nb),
        in_specs=[
            pl.BlockSpec((1, 6, _BA), lambda b, p, j: (b, 0, j)),
            pl.BlockSpec((1, 1, 1, _BA), lambda b, p, j: (b, j, 0, 0)),
            pl.BlockSpec((1, 4, _BA), lambda b, p, j: (b, 0, j)),
            pl.BlockSpec((1, _G, 8), lambda b, p, j: (b, 0, 0)),
        ],
        out_specs=[
            pl.BlockSpec((1, 8, 128), lambda b, p, j: (b, 0, 0)),
            pl.BlockSpec((1, 8, 128), lambda b, p, j: (b, 0, 0)),
        ],
        out_shape=[
            jax.ShapeDtypeStruct((B, 8, 128), jnp.float32),
            jax.ShapeDtypeStruct((B, 8, 128), jnp.float32),
        ],
        scratch_shapes=[
            pltpu.VMEM((nb, 1, _BA), jnp.float32),
            pltpu.VMEM((nb, 1, _BA), jnp.float32),
            pltpu.VMEM((nb, 1, _BA), jnp.float32),
            pltpu.VMEM((nb, 1, _BA), jnp.float32),
            pltpu.VMEM((1, _G), jnp.float32),
            pltpu.VMEM((1, _G), jnp.float32),
            pltpu.SMEM((8,), jnp.float32),
        ],
    )(offs_t, obj_r, anc_t, tgt_r)

    cls_losses = out[0][:, 0, 0]
    regr_losses = out[1][:, 0, 0]
    c = jnp.mean(cls_losses)
    r = jnp.mean(regr_losses)
    return jnp.stack([c + r, c, r])
